# baseline (speedup 1.0000x reference)
import functools

import jax
import jax.numpy as jnp
from jax import lax
from jax.experimental import pallas as pl
from jax.experimental.pallas import tpu as pltpu
from jax.experimental.pallas import tpu_sc as plsc


_SHAPE = (32, 576, 64)
_MESH = plsc.VectorSubcoreMesh(core_axis_name="c", subcore_axis_name="s")


@functools.partial(
    pl.kernel,
    mesh=_MESH,
    out_type=jax.ShapeDtypeStruct(_SHAPE, jnp.float32),
    scratch_types=[
        pltpu.VMEM((1,) + _SHAPE[1:], jnp.float32),
        pltpu.SemaphoreType.DMA,
    ],
)
def _sc_copy(x_hbm, out_hbm, buf, sem):
    wid = lax.axis_index("s") * 2 + lax.axis_index("c")
    pltpu.async_copy(x_hbm.at[pl.ds(wid, 1)], buf, sem).wait()
    pltpu.async_copy(buf, out_hbm.at[pl.ds(wid, 1)], sem).wait()


def kernel(x):
    return _sc_copy(x)


# speedup vs baseline: 1.6758x; 1.6758x over previous
import jax
from jax.experimental import pallas as pl
from jax.experimental.pallas import tpu as pltpu


_N_STRANDS = 8


def _copy_kernel(x_ref, o_ref, vmem, in_sems, out_sems):
    in_copies = [
        pltpu.make_async_copy(x_ref.at[pl.ds(i * 4, 4)], vmem.at[pl.ds(i * 4, 4)], in_sems.at[i])
        for i in range(_N_STRANDS)
    ]
    out_copies = [
        pltpu.make_async_copy(vmem.at[pl.ds(i * 4, 4)], o_ref.at[pl.ds(i * 4, 4)], out_sems.at[i])
        for i in range(_N_STRANDS)
    ]
    for c in in_copies:
        c.start()
    for i in range(_N_STRANDS):
        in_copies[i].wait()
        out_copies[i].start()
    for c in out_copies:
        c.wait()


def kernel(x):
    return pl.pallas_call(
        _copy_kernel,
        out_shape=jax.ShapeDtypeStruct(x.shape, x.dtype),
        in_specs=[pl.BlockSpec(memory_space=pl.ANY)],
        out_specs=pl.BlockSpec(memory_space=pl.ANY),
        scratch_shapes=[
            pltpu.VMEM(x.shape, x.dtype),
            pltpu.SemaphoreType.DMA((_N_STRANDS,)),
            pltpu.SemaphoreType.DMA((_N_STRANDS,)),
        ],
    )(x)
